# baseline (speedup 1.0000x reference)
import functools

import jax
import jax.numpy as jnp
from jax import lax
from jax.experimental import pallas as pl
from jax.experimental.pallas import tpu as pltpu
from jax.experimental.pallas import tpu_sc as plsc

N = 10000
H = 64
HC = 32
NP = 10240
NC = 2
NS = 16
LANES = 16
CH = 128
NBUF = 8
PF = 4
EP = 327680
CPT = EP // (NS * CH)
ROWS_PT = NP // NS

_mesh = plsc.VectorSubcoreMesh(
    core_axis_name="c", subcore_axis_name="s", num_cores=NC, num_subcores=NS
)

_sc_params = pltpu.CompilerParams(
    needs_layout_passes=False, use_tc_tiling_on_sc=False
)


@functools.partial(
    pl.kernel,
    out_type=jax.ShapeDtypeStruct((NC, NP), jnp.float32),
    mesh=_mesh,
    scratch_types=[
        pltpu.VMEM((CPT // NC, CH), jnp.int32),
        pltpu.VMEM((NP,), jnp.float32),
        pltpu.VMEM((ROWS_PT,), jnp.float32),
        pltpu.VMEM((ROWS_PT,), jnp.float32),
        pltpu.VMEM_SHARED((NS, NP), jnp.float32),
    ],
    compiler_params=_sc_params,
)
def _deg_kernel(dst_hbm, out_hbm, idx_v, deg_v, acc_v, tmp_v, part_sh):
    c = lax.axis_index("c")
    s = lax.axis_index("s")
    wid = c * NS + s

    pltpu.sync_copy(dst_hbm.at[pl.ds(wid * (CPT // NC), CPT // NC)], idx_v)

    def zero_body(j, _):
        deg_v[pl.ds(j * LANES, LANES)] = jnp.zeros((LANES,), jnp.float32)
        return 0

    lax.fori_loop(0, NP // LANES, zero_body, 0)

    ones = jnp.ones((LANES,), jnp.float32)

    def chunk_body(t, _):
        for k in range(CH // LANES):
            idx = idx_v[t, pl.ds(k * LANES, LANES)]
            plsc.addupdate_scatter(deg_v, [idx], ones)
        return 0

    lax.fori_loop(0, CPT // NC, chunk_body, 0)

    pltpu.sync_copy(deg_v, part_sh.at[s])
    plsc.subcore_barrier()

    col = s * ROWS_PT
    pltpu.sync_copy(part_sh.at[0, pl.ds(col, ROWS_PT)], acc_v)
    for r in range(1, NS):
        pltpu.sync_copy(part_sh.at[r, pl.ds(col, ROWS_PT)], tmp_v)

        def add_body(j, _):
            sl = pl.ds(j * LANES, LANES)
            acc_v[sl] = acc_v[sl] + tmp_v[sl]
            return 0

        lax.fori_loop(0, ROWS_PT // LANES, add_body, 0)
    pltpu.sync_copy(acc_v, out_hbm.at[c, pl.ds(col, ROWS_PT)])


@functools.partial(
    pl.kernel,
    out_type=jax.ShapeDtypeStruct((NC, NP, HC), jnp.float32),
    mesh=_mesh,
    scratch_types=[
        pltpu.VMEM((CPT, CH), jnp.int32),
        pltpu.VMEM((CPT, CH), jnp.int32),
        [pltpu.VMEM((CH, HC), jnp.float32)] * NBUF,
        pltpu.VMEM_SHARED((NP, HC), jnp.float32),
        pltpu.VMEM_SHARED((NP, HC), jnp.float32),
        [pltpu.SemaphoreType.DMA] * NBUF,
        [pltpu.SemaphoreType.DMA] * NBUF,
    ],
    compiler_params=_sc_params,
)
def _scatter_kernel(src_hbm, dst_hbm, mp_hbm, zeros_hbm, out_hbm,
                    sidx, didx, rows, acc_sh, tbl_sh, gsems, ssems):
    c = lax.axis_index("c")
    s = lax.axis_index("s")
    cb = s * CPT
    row0 = s * ROWS_PT

    pltpu.sync_copy(zeros_hbm.at[pl.ds(row0, ROWS_PT)],
                    acc_sh.at[pl.ds(row0, ROWS_PT)])
    pltpu.sync_copy(mp_hbm.at[c, pl.ds(row0, ROWS_PT)],
                    tbl_sh.at[pl.ds(row0, ROWS_PT)])
    pltpu.sync_copy(src_hbm.at[pl.ds(cb, CPT)], sidx)
    pltpu.sync_copy(dst_hbm.at[pl.ds(cb, CPT)], didx)
    plsc.subcore_barrier()

    for b in range(PF):
        pltpu.async_copy(tbl_sh.at[sidx.at[b]], rows[b], gsems[b])

    def ring_body(g, _):
        for b in range(NBUF):
            t = g + b
            pltpu.make_async_copy(tbl_sh.at[sidx.at[t]], rows[b],
                                  gsems[b]).wait()
            pltpu.async_copy(rows[b], acc_sh.at[didx.at[t]], ssems[b],
                             add=True)
            bf = (b + PF) % NBUF
            tf = t + PF

            @pl.when(tf < CPT)
            def _():
                @pl.when(tf >= NBUF)
                def _():
                    pltpu.make_async_copy(rows[bf], acc_sh.at[didx.at[0]],
                                          ssems[bf]).wait()

                pltpu.async_copy(tbl_sh.at[sidx.at[tf]], rows[bf], gsems[bf])
        return 0

    lax.fori_loop(0, CPT // NBUF, lambda i, z: ring_body(i * NBUF, z), 0)

    for b in range(NBUF):
        pltpu.make_async_copy(rows[b], acc_sh.at[didx.at[0]],
                              ssems[b]).wait()

    plsc.subcore_barrier()
    pltpu.sync_copy(acc_sh.at[pl.ds(row0, ROWS_PT)],
                    out_hbm.at[c, pl.ds(row0, ROWS_PT)])


def _split_store(mp_ref, v):
    mp_ref[0] = v[:, :HC]
    mp_ref[1] = v[:, HC:]


def _t1m_body(x_ref, w1_ref, m_ref):
    m = jnp.dot(x_ref[...], w1_ref[...], preferred_element_type=jnp.float32)
    m_ref[pl.ds(0, N)] = m
    m_ref[pl.ds(N, NP - N)] = jnp.zeros((NP - N, H), jnp.float32)


def _t1s_body(deg_ref, m_ref, dinv_ref, mp_ref):
    deg = deg_ref[0] + deg_ref[1] + 1.0
    dinv = lax.rsqrt(deg)
    dinv_ref[...] = dinv
    _split_store(mp_ref, m_ref[...] * dinv[:, None])


def _t2_body(acc_ref, mp_ref, dinv_ref, b_ref, w_ref, out_ref):
    a = jnp.concatenate(
        [acc_ref[0] + mp_ref[0], acc_ref[1] + mp_ref[1]], axis=1
    )
    dinv = dinv_ref[...]
    h = jnp.maximum(a * dinv[:, None] + b_ref[...][None, :], 0.0)
    _split_store(
        out_ref,
        jnp.dot(h, w_ref[...], preferred_element_type=jnp.float32)
        * dinv[:, None],
    )


def _t3_body(acc_ref, mp_ref, dinv_ref, b_ref, wr_ref, br_ref, out_ref):
    a = jnp.concatenate(
        [acc_ref[0] + mp_ref[0], acc_ref[1] + mp_ref[1]], axis=1
    )
    dinv = dinv_ref[...]
    h = jnp.maximum(a * dinv[:, None] + b_ref[...][None, :], 0.0)
    ridx = lax.broadcasted_iota(jnp.int32, (NP, H), 0)
    h = jnp.where(ridx < N, h, 0.0)
    pooled = jnp.sum(h, axis=0, keepdims=True) * (1.0 / N)
    out_ref[...] = jnp.maximum(
        jnp.dot(pooled, wr_ref[...], preferred_element_type=jnp.float32)
        + br_ref[...][None, :],
        0.0,
    )


_t1m = pl.pallas_call(
    _t1m_body,
    out_shape=jax.ShapeDtypeStruct((NP, H), jnp.float32),
)

_t1s = pl.pallas_call(
    _t1s_body,
    out_shape=(
        jax.ShapeDtypeStruct((NP,), jnp.float32),
        jax.ShapeDtypeStruct((NC, NP, HC), jnp.float32),
    ),
)

_t2 = pl.pallas_call(
    _t2_body,
    out_shape=jax.ShapeDtypeStruct((NC, NP, HC), jnp.float32),
)

_t3 = pl.pallas_call(
    _t3_body,
    out_shape=jax.ShapeDtypeStruct((1, 128), jnp.float32),
)


def kernel(x, edge_index, W1, b1, W2, b2, W3, b3, Wr, br):
    e = edge_index.shape[1]
    pad = jnp.full((EP - e,), N, jnp.int32)
    src = jnp.concatenate([edge_index[0], pad]).reshape(NS * CPT, CH)
    dst = jnp.concatenate([edge_index[1], pad]).reshape(NS * CPT, CH)
    zeros2d = jnp.zeros((NP, HC), jnp.float32)

    deg = _deg_kernel(dst)
    m1 = _t1m(x, W1)
    dinv, mp1 = _t1s(deg, m1)
    acc1 = _scatter_kernel(src, dst, mp1, zeros2d)
    mp2 = _t2(acc1, mp1, dinv, b1, W2)
    acc2 = _scatter_kernel(src, dst, mp2, zeros2d)
    mp3 = _t2(acc2, mp2, dinv, b2, W3)
    acc3 = _scatter_kernel(src, dst, mp3, zeros2d)
    out = _t3(acc3, mp3, dinv, b3, Wr, br)
    return out


# speedup vs baseline: 1.2669x; 1.2669x over previous
import functools

import jax
import jax.numpy as jnp
from jax import lax
from jax.experimental import pallas as pl
from jax.experimental.pallas import tpu as pltpu
from jax.experimental.pallas import tpu_sc as plsc

N = 10000
H = 64
HC = 32
NP = 10240
NC = 2
NS = 16
LANES = 16
CH = 128
NBUF = 8
PF = 4
EP = 327680
CPT = EP // (NS * CH)
ROWS_PT = NP // NS

_mesh = plsc.VectorSubcoreMesh(
    core_axis_name="c", subcore_axis_name="s", num_cores=NC, num_subcores=NS
)

_sc_params = pltpu.CompilerParams(
    needs_layout_passes=False, use_tc_tiling_on_sc=False
)


@functools.partial(
    pl.kernel,
    out_type=jax.ShapeDtypeStruct((NC, NP), jnp.float32),
    mesh=_mesh,
    scratch_types=[
        pltpu.VMEM((CPT // NC, CH), jnp.int32),
        pltpu.VMEM((NP,), jnp.float32),
        pltpu.VMEM((ROWS_PT,), jnp.float32),
        pltpu.VMEM((ROWS_PT,), jnp.float32),
        pltpu.VMEM_SHARED((NS, NP), jnp.float32),
    ],
    compiler_params=_sc_params,
)
def _deg_kernel(dst_hbm, out_hbm, idx_v, deg_v, acc_v, tmp_v, part_sh):
    c = lax.axis_index("c")
    s = lax.axis_index("s")
    wid = c * NS + s

    pltpu.sync_copy(dst_hbm.at[pl.ds(wid * (CPT // NC), CPT // NC)], idx_v)

    def zero_body(j, _):
        deg_v[pl.ds(j * LANES, LANES)] = jnp.zeros((LANES,), jnp.float32)
        return 0

    lax.fori_loop(0, NP // LANES, zero_body, 0)

    ones = jnp.ones((LANES,), jnp.float32)

    def chunk_body(t, _):
        for k in range(CH // LANES):
            idx = idx_v[t, pl.ds(k * LANES, LANES)]
            plsc.addupdate_scatter(deg_v, [idx], ones)
        return 0

    lax.fori_loop(0, CPT // NC, chunk_body, 0)

    pltpu.sync_copy(deg_v, part_sh.at[s])
    plsc.subcore_barrier()

    col = s * ROWS_PT
    pltpu.sync_copy(part_sh.at[0, pl.ds(col, ROWS_PT)], acc_v)
    for r in range(1, NS):
        pltpu.sync_copy(part_sh.at[r, pl.ds(col, ROWS_PT)], tmp_v)

        def add_body(j, _):
            sl = pl.ds(j * LANES, LANES)
            acc_v[sl] = acc_v[sl] + tmp_v[sl]
            return 0

        lax.fori_loop(0, ROWS_PT // LANES, add_body, 0)
    pltpu.sync_copy(acc_v, out_hbm.at[c, pl.ds(col, ROWS_PT)])


@functools.partial(
    pl.kernel,
    out_type=jax.ShapeDtypeStruct((NC, NP, HC), jnp.float32),
    mesh=_mesh,
    scratch_types=[
        pltpu.VMEM((CPT, CH), jnp.int32),
        pltpu.VMEM((CPT, CH), jnp.int32),
        [pltpu.VMEM((CH, HC), jnp.float32)] * NBUF,
        pltpu.VMEM_SHARED((NP, HC), jnp.float32),
        pltpu.VMEM_SHARED((NP, HC), jnp.float32),
        [pltpu.SemaphoreType.DMA] * NBUF,
        [pltpu.SemaphoreType.DMA] * NBUF,
    ],
    compiler_params=_sc_params,
)
def _scatter_kernel(src_hbm, dst_hbm, mp_hbm, zeros_hbm, out_hbm,
                    sidx, didx, rows, acc_sh, tbl_sh, gsems, ssems):
    c = lax.axis_index("c")
    s = lax.axis_index("s")
    cb = s * CPT
    row0 = s * ROWS_PT

    pltpu.sync_copy(zeros_hbm.at[pl.ds(row0, ROWS_PT)],
                    acc_sh.at[pl.ds(row0, ROWS_PT)])
    pltpu.sync_copy(mp_hbm.at[c, pl.ds(row0, ROWS_PT)],
                    tbl_sh.at[pl.ds(row0, ROWS_PT)])
    pltpu.sync_copy(src_hbm.at[pl.ds(cb, CPT)], sidx)
    pltpu.sync_copy(dst_hbm.at[pl.ds(cb, CPT)], didx)
    plsc.subcore_barrier()

    for b in range(PF):
        pltpu.async_copy(tbl_sh.at[sidx.at[b]], rows[b], gsems[b])

    def ring_body(g, _):
        for b in range(NBUF):
            t = g + b
            pltpu.make_async_copy(tbl_sh.at[sidx.at[t]], rows[b],
                                  gsems[b]).wait()
            pltpu.async_copy(rows[b], acc_sh.at[didx.at[t]], ssems[b],
                             add=True)
            bf = (b + PF) % NBUF
            tf = t + PF

            @pl.when(tf < CPT)
            def _():
                @pl.when(tf >= NBUF)
                def _():
                    pltpu.make_async_copy(rows[bf], acc_sh.at[didx.at[0]],
                                          ssems[bf]).wait()

                pltpu.async_copy(tbl_sh.at[sidx.at[tf]], rows[bf], gsems[bf])
        return 0

    lax.fori_loop(0, CPT // NBUF, lambda i, z: ring_body(i * NBUF, z), 0)

    for b in range(NBUF):
        pltpu.make_async_copy(rows[b], acc_sh.at[didx.at[0]],
                              ssems[b]).wait()

    plsc.subcore_barrier()
    pltpu.sync_copy(acc_sh.at[pl.ds(row0, ROWS_PT)],
                    out_hbm.at[c, pl.ds(row0, ROWS_PT)])


RP = NP // 4
RN = N // 4


def _dinv_packed(deg4):
    dinv4 = lax.rsqrt(deg4 + 1.0)
    return jnp.concatenate(
        [jnp.broadcast_to(dinv4[:, j:j + 1], (RP, HC)) for j in range(4)],
        axis=1,
    )


def _t1_body(deg4_ref, xp_ref, bw1_ref, dinvp_ref, mpp_ref):
    dinvp = _dinv_packed(deg4_ref[0] + deg4_ref[1])
    dinvp_ref[...] = dinvp
    for c in range(NC):
        m = jnp.dot(xp_ref[...], bw1_ref[c],
                    preferred_element_type=jnp.float32)
        mpp_ref[c, pl.ds(0, RN)] = m * dinvp[:RN]
        mpp_ref[c, pl.ds(RN, RP - RN)] = jnp.zeros((RP - RN, 4 * HC),
                                                   jnp.float32)


def _t2_body(accp_ref, mpp_ref, dinvp_ref, bp_ref, bw_ref, outp_ref):
    dinvp = dinvp_ref[...]
    h = [
        jnp.maximum(
            (accp_ref[c] + mpp_ref[c]) * dinvp + bp_ref[c][None, :], 0.0
        )
        for c in range(NC)
    ]
    for co in range(NC):
        m = (
            jnp.dot(h[0], bw_ref[0, co], preferred_element_type=jnp.float32)
            + jnp.dot(h[1], bw_ref[1, co], preferred_element_type=jnp.float32)
        )
        outp_ref[co] = m * dinvp


def _t3_body(accp_ref, mpp_ref, dinvp_ref, bp_ref, wr_ref, br_ref, out_ref):
    dinvp = dinvp_ref[...]
    ridx = lax.broadcasted_iota(jnp.int32, (RP, 4 * HC), 0)
    pools = []
    for c in range(NC):
        h = jnp.maximum(
            (accp_ref[c] + mpp_ref[c]) * dinvp + bp_ref[c][None, :], 0.0
        )
        h = jnp.where(ridx < RN, h, 0.0)
        cs = jnp.sum(h, axis=0)
        pools.append(cs[0:HC] + cs[HC:2 * HC] + cs[2 * HC:3 * HC]
                     + cs[3 * HC:4 * HC])
    pooled = jnp.concatenate(pools)[None, :] * (1.0 / N)
    out_ref[...] = jnp.maximum(
        jnp.dot(pooled, wr_ref[...], preferred_element_type=jnp.float32)
        + br_ref[...][None, :],
        0.0,
    )


_t1 = pl.pallas_call(
    _t1_body,
    out_shape=(
        jax.ShapeDtypeStruct((RP, 4 * HC), jnp.float32),
        jax.ShapeDtypeStruct((NC, RP, 4 * HC), jnp.float32),
    ),
)

_t2 = pl.pallas_call(
    _t2_body,
    out_shape=jax.ShapeDtypeStruct((NC, RP, 4 * HC), jnp.float32),
)

_t3 = pl.pallas_call(
    _t3_body,
    out_shape=jax.ShapeDtypeStruct((1, 128), jnp.float32),
)

_EYE4 = None


def _pack_w_in(W):
    eye4 = jnp.eye(4, dtype=jnp.float32)
    return jnp.stack(
        [jnp.kron(eye4, W[:, c * HC:(c + 1) * HC]) for c in range(NC)]
    )


def _pack_w_mid(W):
    eye4 = jnp.eye(4, dtype=jnp.float32)
    return jnp.stack([
        jnp.stack([
            jnp.kron(eye4, W[ch * HC:(ch + 1) * HC, co * HC:(co + 1) * HC])
            for co in range(NC)
        ])
        for ch in range(NC)
    ])


def _pack_b(b):
    return jnp.stack(
        [jnp.tile(b[c * HC:(c + 1) * HC], 4) for c in range(NC)]
    )


def kernel(x, edge_index, W1, b1, W2, b2, W3, b3, Wr, br):
    e = edge_index.shape[1]
    pad = jnp.full((EP - e,), N, jnp.int32)
    src = jnp.concatenate([edge_index[0], pad]).reshape(NS * CPT, CH)
    dst = jnp.concatenate([edge_index[1], pad]).reshape(NS * CPT, CH)
    zeros2d = jnp.zeros((NP, HC), jnp.float32)

    xp = x.reshape(RN, 4 * 128)
    bw1 = _pack_w_in(W1)
    bw2 = _pack_w_mid(W2)
    bw3 = _pack_w_mid(W3)
    b1p, b2p, b3p = _pack_b(b1), _pack_b(b2), _pack_b(b3)

    deg = _deg_kernel(dst)
    deg4 = deg.reshape(NC, RP, 4)
    dinvp, mpp1 = _t1(deg4, xp, bw1)
    acc1 = _scatter_kernel(src, dst, mpp1.reshape(NC, NP, HC), zeros2d)
    mpp2 = _t2(acc1.reshape(NC, RP, 4 * HC), mpp1, dinvp, b1p, bw2)
    acc2 = _scatter_kernel(src, dst, mpp2.reshape(NC, NP, HC), zeros2d)
    mpp3 = _t2(acc2.reshape(NC, RP, 4 * HC), mpp2, dinvp, b2p, bw3)
    acc3 = _scatter_kernel(src, dst, mpp3.reshape(NC, NP, HC), zeros2d)
    out = _t3(acc3.reshape(NC, RP, 4 * HC), mpp3, dinvp, b3p, Wr, br)
    return out
